# Initial kernel scaffold; baseline (speedup 1.0000x reference)
import jax
import jax.numpy as jnp
from jax.experimental import pallas as pl


def kernel(a1, w1, w2, topk_weights, topk_ids):
    raise NotImplementedError("write your pallas kernel here")


# speedup vs baseline: 2.4761x; 2.4761x over previous
import functools

import jax
import jax.numpy as jnp
from jax import lax
from jax.experimental import pallas as pl
from jax.experimental.pallas import tpu as pltpu

M, K, N, E, TOPK = 2048, 1024, 1024, 8, 2


def _moe_dense_body(a_ref, w1_ref, w2_ref, tw_ref, ids_ref, out_ref):
    e = pl.program_id(0)
    a = a_ref[...]
    w1e = w1_ref[0]
    w2e = w2_ref[0]
    h = lax.dot_general(a, w1e, (((1,), (1,)), ((), ())),
                        preferred_element_type=jnp.float32)
    gate = h[:, :N]
    up = h[:, N:]
    act = (gate * jax.nn.sigmoid(gate)) * up
    o = lax.dot_general(act, w2e, (((1,), (1,)), ((), ())),
                        preferred_element_type=jnp.float32)
    w = jnp.sum(tw_ref[...] * (ids_ref[...] == e).astype(jnp.float32),
                axis=1, keepdims=True)
    contrib = w * o

    @pl.when(e == 0)
    def _():
        out_ref[...] = contrib

    @pl.when(e > 0)
    def _():
        out_ref[...] += contrib


@jax.jit
def kernel(a1, w1, w2, topk_weights, topk_ids):
    ids = topk_ids.astype(jnp.int32)
    out = pl.pallas_call(
        _moe_dense_body,
        grid=(E,),
        in_specs=[
            pl.BlockSpec((M, K), lambda e: (0, 0)),
            pl.BlockSpec((1, 2 * N, K), lambda e: (e, 0, 0)),
            pl.BlockSpec((1, K, N), lambda e: (e, 0, 0)),
            pl.BlockSpec((M, TOPK), lambda e: (0, 0)),
            pl.BlockSpec((M, TOPK), lambda e: (0, 0)),
        ],
        out_specs=pl.BlockSpec((M, K), lambda e: (0, 0)),
        out_shape=jax.ShapeDtypeStruct((M, K), jnp.float32),
        compiler_params=pltpu.CompilerParams(
            dimension_semantics=("arbitrary",),
        ),
    )(a1, w1, w2, topk_weights, ids)
    return out
